# Initial kernel scaffold; baseline (speedup 1.0000x reference)
import jax
import jax.numpy as jnp
from jax.experimental import pallas as pl


def kernel(x, edge_index, W, b):
    raise NotImplementedError("write your pallas kernel here")


# speedup vs baseline: 1.1327x; 1.1327x over previous
import functools

import jax
import jax.numpy as jnp
from jax import lax
from jax.experimental import pallas as pl
from jax.experimental.pallas import tpu as pltpu
from jax.experimental.pallas import tpu_sc as plsc

N = 10000
D = 128
NE = 320000

_info = plsc.get_sparse_core_info()
NC = _info.num_cores
NS = _info.num_subcores
NW = NC * NS
NPT = 320
NPAD = NW * NPT
CHUNK = 6400
NCHUNK = NE // CHUNK
G = 64
ACH = 32
LANES = 16


def _mm_body(x_ref, w_ref, b_ref, a_ref, bm_ref):
    xb = x_ref[...]
    w1 = w_ref[:D, :]
    w2 = w_ref[D:, :]
    bm_ref[...] = jnp.dot(xb, w2, preferred_element_type=jnp.float32)
    a_ref[...] = (
        jnp.dot(xb, w1 - w2, preferred_element_type=jnp.float32) + b_ref[...]
    )


_BLKM = 2048


def _matmuls(xp, W, b2):
    return pl.pallas_call(
        _mm_body,
        grid=(NPAD // _BLKM,),
        in_specs=[
            pl.BlockSpec((_BLKM, D), lambda i: (i, 0)),
            pl.BlockSpec((2 * D, D), lambda i: (0, 0)),
            pl.BlockSpec((1, D), lambda i: (0, 0)),
        ],
        out_specs=[
            pl.BlockSpec((_BLKM, D), lambda i: (i, 0)),
            pl.BlockSpec((_BLKM, D), lambda i: (i, 0)),
        ],
        out_shape=[
            jax.ShapeDtypeStruct((NPAD, D), jnp.float32),
            jax.ShapeDtypeStruct((NPAD, D), jnp.float32),
        ],
    )(xp, W, b2)


def _sc_body(bm_hbm, a_hbm, src_hbm, dst_hbm, out_hbm,
             dstb, srcb, csrc, cdst, accum, rows, abuf, sem):
    wid = lax.axis_index("s") * NC + lax.axis_index("c")
    lo = wid * NPT

    neg = jnp.full((LANES,), -jnp.inf, jnp.float32)

    def init_body(i, _):
        accum[pl.ds(i * LANES, LANES)] = neg
        return 0

    lax.fori_loop(0, (NPT + 1) * D // LANES, init_body, 0, unroll=4)

    iota = lax.iota(jnp.int32, LANES)
    one = jnp.ones((LANES,), jnp.int32)
    zero16 = jnp.zeros((LANES,), jnp.int32)
    pad16 = jnp.full((LANES,), NPT, jnp.int32)

    def chunk_body(c, _):
        off = pl.multiple_of(c * CHUNK, CHUNK)
        pltpu.sync_copy(dst_hbm.at[pl.ds(off, CHUNK)], dstb)
        pltpu.sync_copy(src_hbm.at[pl.ds(off, CHUNK)], srcb)

        def scan_body(i, nm):
            d = dstb[pl.ds(i * LANES, LANES)]
            s = srcb[pl.ds(i * LANES, LANES)]
            dl = d - lo
            m = (dl >= 0) & (dl < NPT)
            pos = plsc.cumsum(jnp.where(m, one, zero16)) + nm - 1
            plsc.store_scatter(csrc, [pos], s, mask=m)
            plsc.store_scatter(cdst, [pos], dl, mask=m)
            return nm + plsc.all_reduce_population_count(m)

        nmv = lax.fori_loop(0, CHUNK // LANES, scan_body,
                            jnp.zeros((LANES,), jnp.int32), unroll=4)
        for kpad in range(G // LANES):
            posp = nmv + (kpad * LANES) + iota
            plsc.store_scatter(csrc, [posp], zero16)
            plsc.store_scatter(cdst, [posp], pad16)
        nm = jnp.max(nmv)
        ngrp = (nm + (G - 1)) >> 6

        def grp_body(g, _):
            goff = g * G
            idx = csrc.at[pl.ds(goff, G)]
            pltpu.async_copy(bm_hbm.at[idx], rows, sem).wait()

            def row_body(r, _):
                ld = cdst[pl.ds(goff + r, LANES)][0]
                rb = ld << 7
                for k in range(D // LANES):
                    a = accum[pl.ds(rb + k * LANES, LANES)]
                    v = rows[r, pl.ds(k * LANES, LANES)]
                    accum[pl.ds(rb + k * LANES, LANES)] = jnp.maximum(a, v)
                return 0

            lax.fori_loop(0, G, row_body, 0)
            return 0

        lax.fori_loop(0, ngrp, grp_body, 0)
        return 0

    lax.fori_loop(0, NCHUNK, chunk_body, 0)

    def comb_body(t, _):
        row0 = lo + t * ACH
        pltpu.sync_copy(a_hbm.at[pl.ds(row0, ACH)], abuf)

        def cr(i, _):
            rb = (t * ACH + i) * D
            for k in range(D // LANES):
                av = abuf[i, pl.ds(k * LANES, LANES)]
                mv = accum[pl.ds(rb + k * LANES, LANES)]
                abuf[i, pl.ds(k * LANES, LANES)] = jnp.maximum(av + mv, 0.0)
            return 0

        lax.fori_loop(0, ACH, cr, 0)
        pltpu.sync_copy(abuf, out_hbm.at[pl.ds(row0, ACH)])
        return 0

    lax.fori_loop(0, NPT // ACH, comb_body, 0)


_sc_call = functools.partial(
    pl.kernel,
    out_type=jax.ShapeDtypeStruct((NPAD, D), jnp.float32),
    mesh=plsc.VectorSubcoreMesh(core_axis_name="c", subcore_axis_name="s"),
    scratch_types=[
        pltpu.VMEM((CHUNK,), jnp.int32),
        pltpu.VMEM((CHUNK,), jnp.int32),
        pltpu.VMEM((CHUNK + G,), jnp.int32),
        pltpu.VMEM((CHUNK + G + LANES,), jnp.int32),
        pltpu.VMEM(((NPT + 1) * D,), jnp.float32),
        pltpu.VMEM((G, D), jnp.float32),
        pltpu.VMEM((ACH, D), jnp.float32),
        pltpu.SemaphoreType.DMA,
    ],
    compiler_params=pltpu.CompilerParams(needs_layout_passes=False),
)(_sc_body)


@jax.jit
def kernel(x, edge_index, W, b):
    ei = edge_index.astype(jnp.int32)
    src = ei[0]
    dst = ei[1]
    xp = jnp.pad(x, ((0, NPAD - N), (0, 0)))
    aprime, bmat = _matmuls(xp, W, b.reshape(1, D))
    out = _sc_call(bmat, aprime, src, dst)
    return out[:N]
